# baseline (speedup 1.0000x reference)
import functools

import jax
import jax.numpy as jnp
from jax import lax
from jax.experimental import pallas as pl
from jax.experimental.pallas import tpu as pltpu
from jax.experimental.pallas import tpu_sc as plsc

N_NODES = 10000
N_EDGES = 320000
D_FEAT = 128

NC = 2
NS = 16
NW = NC * NS

CH = 128
N_CHUNKS = N_EDGES // CH
BASE_CH = N_CHUNKS // NW
REM = N_CHUNKS - BASE_CH * NW
IDX_ROWS = BASE_CH + 1
RB = 2
IB = 4
N_OUTER = -(-(IDX_ROWS + 1) // IB)

ROW_SLICE = 624
ROW_REM = N_NODES - ROW_SLICE * NS

_MESH = plsc.VectorSubcoreMesh(
    core_axis_name="c", subcore_axis_name="s", num_cores=NC, num_subcores=NS
)


def _worker_range(w):
    start = BASE_CH * w + jnp.maximum(w - (NW - REM), 0)
    n = BASE_CH + (w >= NW - REM).astype(jnp.int32)
    return start, n


def _guarded(cond, fn, *args):
    @pl.when(cond)
    def _():
        fn(*args)


def _copy_row_slices(src, dst, s):
    pltpu.sync_copy(src.at[pl.ds(s * ROW_SLICE, ROW_SLICE)],
                    dst.at[pl.ds(s * ROW_SLICE, ROW_SLICE)])

    @pl.when(s == NS - 1)
    def _():
        pltpu.sync_copy(src.at[pl.ds(ROW_SLICE * NS, ROW_REM)],
                        dst.at[pl.ds(ROW_SLICE * NS, ROW_REM)])


def _repack_rows(flat, two_d, n_rows):
    def body(j, carry):
        for k in range(CH // 16):
            two_d[j, pl.ds(k * 16, 16)] = flat[pl.ds(j * CH + k * 16, 16)]
        return carry

    lax.fori_loop(0, n_rows, body, 0)


_DEG_CAP = 8


def _deg_body(edges_hbm, zeros_hbm, degp_hbm,
              idx_flat, idxs2, idxd2, ones_v, sem_i, sem_s, dout_sh, din_sh):
    c = lax.axis_index("c")
    s = lax.axis_index("s")
    w = s * NC + c
    start_w, n_w = _worker_range(w)

    @pl.when(s == 0)
    def _():
        pltpu.sync_copy(zeros_hbm, dout_sh)
        pltpu.sync_copy(zeros_hbm, din_sh)

    for i in range(CH // 16):
        ones_v[pl.ds(i * 16, 16)] = jnp.ones((16,), jnp.float32)

    pltpu.async_copy(edges_hbm.at[0, pl.ds(start_w * CH, IDX_ROWS * CH)],
                     idx_flat, sem_i)
    pltpu.make_async_copy(edges_hbm.at[0, pl.ds(0, IDX_ROWS * CH)],
                          idx_flat, sem_i).wait()
    _repack_rows(idx_flat, idxs2, n_w)
    pltpu.async_copy(edges_hbm.at[1, pl.ds(start_w * CH, IDX_ROWS * CH)],
                     idx_flat, sem_i)
    pltpu.make_async_copy(edges_hbm.at[0, pl.ds(0, IDX_ROWS * CH)],
                          idx_flat, sem_i).wait()
    _repack_rows(idx_flat, idxd2, n_w)

    plsc.subcore_barrier()

    def wait_two():
        pltpu.make_async_copy(ones_v, dout_sh.at[idxs2.at[0]], sem_s).wait()
        pltpu.make_async_copy(ones_v, din_sh.at[idxd2.at[0]], sem_s).wait()

    def body(j, carry):
        _guarded(j >= _DEG_CAP, wait_two)
        pltpu.async_copy(ones_v, dout_sh.at[idxs2.at[j]], sem_s, add=True)
        pltpu.async_copy(ones_v, din_sh.at[idxd2.at[j]], sem_s, add=True)
        return carry

    lax.fori_loop(0, n_w, body, 0)
    for _ in range(_DEG_CAP):
        wait_two()

    plsc.subcore_barrier()

    @pl.when(s == 0)
    def _():
        pltpu.sync_copy(dout_sh, degp_hbm.at[c, 0])
        pltpu.sync_copy(din_sh, degp_hbm.at[c, 1])


_deg_call = functools.partial(
    pl.kernel,
    mesh=_MESH,
    out_type=jax.ShapeDtypeStruct((NC, 2, N_NODES), jnp.float32),
    scratch_types=[
        pltpu.VMEM((IDX_ROWS * CH,), jnp.int32),
        pltpu.VMEM((IDX_ROWS, CH), jnp.int32),
        pltpu.VMEM((IDX_ROWS, CH), jnp.int32),
        pltpu.VMEM((CH,), jnp.float32),
        pltpu.SemaphoreType.DMA,
        pltpu.SemaphoreType.DMA,
        pltpu.VMEM_SHARED((N_NODES,), jnp.float32),
        pltpu.VMEM_SHARED((N_NODES,), jnp.float32),
    ],
)(_deg_body)


def _agg_body(h_hbm, edges_hbm, zeros_hbm, aggp_hbm,
              idxs_b, idxd_b, rows_v, sem_i, sem_g, sem_s, agg_sh):
    c = lax.axis_index("c")
    s = lax.axis_index("s")
    w = s * NC + c
    start_w, n_w = _worker_range(w)

    def issue_idx(j, q):
        base = (start_w + j) * CH
        pltpu.async_copy(edges_hbm.at[0, pl.ds(base, CH)], idxs_b.at[q],
                         sem_i.at[q])
        pltpu.async_copy(edges_hbm.at[1, pl.ds(base, CH)], idxd_b.at[q],
                         sem_i.at[q])

    def wait_idx(q):
        pltpu.make_async_copy(edges_hbm.at[0, pl.ds(0, CH)], idxs_b.at[q],
                              sem_i.at[q]).wait()
        pltpu.make_async_copy(edges_hbm.at[0, pl.ds(0, CH)], idxd_b.at[q],
                              sem_i.at[q]).wait()

    def wait_scatter(r):
        pltpu.make_async_copy(rows_v.at[r], agg_sh.at[idxd_b.at[0]],
                              sem_s.at[r]).wait()

    HCH = CH // 2

    def issue_gather(q, r):
        pltpu.async_copy(h_hbm.at[idxs_b.at[q, pl.ds(0, HCH)]],
                         rows_v.at[r, pl.ds(0, HCH)], sem_g.at[r])
        pltpu.async_copy(h_hbm.at[idxs_b.at[q, pl.ds(HCH, HCH)]],
                         rows_v.at[r, pl.ds(HCH, HCH)], sem_g.at[r])

    def wait_gather(r):
        pltpu.make_async_copy(h_hbm.at[idxs_b.at[0, pl.ds(0, HCH)]],
                              rows_v.at[r, pl.ds(0, HCH)], sem_g.at[r]).wait()
        pltpu.make_async_copy(h_hbm.at[idxs_b.at[0, pl.ds(0, HCH)]],
                              rows_v.at[r, pl.ds(0, HCH)], sem_g.at[r]).wait()

    def next_gather(j, u):
        r1 = (u + 1) % RB
        wait_idx((u + 1) % IB)
        issue_gather((u + 1) % IB, r1)

    def chunk_step(j, u):
        r = u % RB
        _guarded(j >= 1, wait_scatter, (u + 1) % RB)
        _guarded(j + RB < n_w, issue_idx, j + RB, (u + RB) % IB)
        _guarded(j + 1 < n_w, next_gather, j, u)
        wait_gather(r)
        pltpu.async_copy(rows_v.at[r], agg_sh.at[idxd_b.at[u]],
                         sem_s.at[r], add=True)

    for j0 in range(RB):
        issue_idx(j0, j0)
    _copy_row_slices(zeros_hbm, agg_sh, s)
    plsc.subcore_barrier()
    wait_idx(0)
    issue_gather(0, 0)

    def outer(t, carry):
        for u in range(IB):
            j = t * IB + u
            _guarded(j < n_w, chunk_step, j, u)
        return carry

    lax.fori_loop(0, N_OUTER, outer, 0)
    _guarded(lax.rem(n_w, 2) == 1, wait_scatter, 0)
    _guarded(lax.rem(n_w, 2) == 0, wait_scatter, 1)

    plsc.subcore_barrier()

    _copy_row_slices(agg_sh, aggp_hbm.at[c], s)


_agg_call = functools.partial(
    pl.kernel,
    mesh=_MESH,
    out_type=jax.ShapeDtypeStruct((NC, N_NODES, D_FEAT), jnp.float32),
    scratch_types=[
        pltpu.VMEM((IB, CH), jnp.int32),
        pltpu.VMEM((IB, CH), jnp.int32),
        pltpu.VMEM((RB, CH, D_FEAT), jnp.float32),
        pltpu.SemaphoreType.DMA((IB,)),
        pltpu.SemaphoreType.DMA((RB,)),
        pltpu.SemaphoreType.DMA((RB,)),
        pltpu.VMEM_SHARED((N_NODES, D_FEAT), jnp.float32),
    ],
)(_agg_body)


_BR = 1000


def _prep_body(degp_ref, feat_ref, h1_ref, scales_ref):
    d = degp_ref[...]
    deg = jnp.maximum(d[:, :, 0] + d[:, :, 1], 1.0)
    sc = lax.rsqrt(deg)
    scales_ref[...] = sc
    h1_ref[...] = feat_ref[...] * sc[:, 0:1]


def _prep_call(degp_t, features):
    return pl.pallas_call(
        _prep_body,
        grid=(N_NODES // _BR,),
        in_specs=[
            pl.BlockSpec((_BR, 2, NC), lambda i: (i, 0, 0)),
            pl.BlockSpec((_BR, D_FEAT), lambda i: (i, 0)),
        ],
        out_specs=[
            pl.BlockSpec((_BR, D_FEAT), lambda i: (i, 0)),
            pl.BlockSpec((_BR, 2), lambda i: (i, 0)),
        ],
        out_shape=[
            jax.ShapeDtypeStruct((N_NODES, D_FEAT), jnp.float32),
            jax.ShapeDtypeStruct((N_NODES, 2), jnp.float32),
        ],
    )(degp_t, features)


def _layer_body(scale_out, aggp_ref, scales_ref, w_ref, b_ref, out_ref):
    a = aggp_ref[0] + aggp_ref[1]
    x = a * scales_ref[:, 1:2]
    y = jnp.dot(x, w_ref[...], preferred_element_type=jnp.float32) + b_ref[...]
    z = jnp.maximum(y, 0.01 * y)
    if scale_out:
        z = z * scales_ref[:, 0:1]
    out_ref[...] = z


def _layer_call(aggp, scales, W, b2d, scale_out):
    return pl.pallas_call(
        functools.partial(_layer_body, scale_out),
        grid=(N_NODES // _BR,),
        in_specs=[
            pl.BlockSpec((NC, _BR, D_FEAT), lambda i: (0, i, 0)),
            pl.BlockSpec((_BR, 2), lambda i: (i, 0)),
            pl.BlockSpec((D_FEAT, D_FEAT), lambda i: (0, 0)),
            pl.BlockSpec((1, D_FEAT), lambda i: (0, 0)),
        ],
        out_specs=pl.BlockSpec((_BR, D_FEAT), lambda i: (i, 0)),
        out_shape=jax.ShapeDtypeStruct((N_NODES, D_FEAT), jnp.float32),
    )(aggp, scales, W, b2d)


def kernel(features, edge_index, W1, b1, W2, b2):
    edges = edge_index.astype(jnp.int32)
    zeros1 = jnp.zeros((N_NODES,), jnp.float32)
    zeros2 = jnp.zeros((N_NODES, D_FEAT), jnp.float32)

    degp = _deg_call(edges, zeros1)
    degp_t = jnp.transpose(degp, (2, 1, 0))
    h1, scales = _prep_call(degp_t, features)

    aggp1 = _agg_call(h1, edges, zeros2)
    h2 = _layer_call(aggp1, scales, W1, b1.reshape(1, D_FEAT), True)

    aggp2 = _agg_call(h2, edges, zeros2)
    out = _layer_call(aggp2, scales, W2, b2.reshape(1, D_FEAT), False)
    return out


# speedup vs baseline: 1.0656x; 1.0656x over previous
import functools

import jax
import jax.numpy as jnp
from jax import lax
from jax.experimental import pallas as pl
from jax.experimental.pallas import tpu as pltpu
from jax.experimental.pallas import tpu_sc as plsc

N_NODES = 10000
N_EDGES = 320000
D_FEAT = 128

NC = 2
NS = 16
NW = NC * NS

CH = 128
HCH = CH // 2
N_CHUNKS = N_EDGES // CH
BASE_CH = N_CHUNKS // NW
REM = N_CHUNKS - BASE_CH * NW
IDX_ROWS = BASE_CH + 1
RB = 2
IB = 4
N_OUTER = -(-(IDX_ROWS + 1) // IB)

ROW_SLICE = 624
ROW_REM = N_NODES - ROW_SLICE * NS

_MESH = plsc.VectorSubcoreMesh(
    core_axis_name="c", subcore_axis_name="s", num_cores=NC, num_subcores=NS
)


def _worker_range(w):
    start = BASE_CH * w + jnp.maximum(w - (NW - REM), 0)
    n = BASE_CH + (w >= NW - REM).astype(jnp.int32)
    return start, n


def _guarded(cond, fn, *args):
    @pl.when(cond)
    def _():
        fn(*args)


def _copy_row_slices(src, dst, s):
    pltpu.sync_copy(src.at[pl.ds(s * ROW_SLICE, ROW_SLICE)],
                    dst.at[pl.ds(s * ROW_SLICE, ROW_SLICE)])

    @pl.when(s == NS - 1)
    def _():
        pltpu.sync_copy(src.at[pl.ds(ROW_SLICE * NS, ROW_REM)],
                        dst.at[pl.ds(ROW_SLICE * NS, ROW_REM)])


def _repack_rows(flat, two_d, n_rows):
    def body(j, carry):
        for k in range(CH // 16):
            two_d[j, pl.ds(k * 16, 16)] = flat[pl.ds(j * CH + k * 16, 16)]
        return carry

    lax.fori_loop(0, n_rows, body, 0)


def _rsqrt16(d):
    d = jnp.maximum(d, 1.0)
    i = lax.bitcast_convert_type(d, jnp.int32)
    i = jnp.int32(0x5F3759DF) - (i >> 1)
    y = lax.bitcast_convert_type(i, jnp.float32)
    for _ in range(3):
        y = y * (1.5 - 0.5 * d * y * y)
    return y


_DEG_CAP = 8


def _deg_body(edges_hbm, zeros_hbm, degp_hbm,
              idx_flat, idxs2, idxd2, ones_v, sem_i, sem_s, dout_sh, din_sh):
    c = lax.axis_index("c")
    s = lax.axis_index("s")
    w = s * NC + c
    start_w, n_w = _worker_range(w)

    @pl.when(s == 0)
    def _():
        pltpu.sync_copy(zeros_hbm, dout_sh)
        pltpu.sync_copy(zeros_hbm, din_sh)

    for i in range(CH // 16):
        ones_v[pl.ds(i * 16, 16)] = jnp.ones((16,), jnp.float32)

    pltpu.async_copy(edges_hbm.at[0, pl.ds(start_w * CH, IDX_ROWS * CH)],
                     idx_flat, sem_i)
    pltpu.make_async_copy(edges_hbm.at[0, pl.ds(0, IDX_ROWS * CH)],
                          idx_flat, sem_i).wait()
    _repack_rows(idx_flat, idxs2, n_w)
    pltpu.async_copy(edges_hbm.at[1, pl.ds(start_w * CH, IDX_ROWS * CH)],
                     idx_flat, sem_i)
    pltpu.make_async_copy(edges_hbm.at[0, pl.ds(0, IDX_ROWS * CH)],
                          idx_flat, sem_i).wait()
    _repack_rows(idx_flat, idxd2, n_w)

    plsc.subcore_barrier()

    def wait_two():
        pltpu.make_async_copy(ones_v, dout_sh.at[idxs2.at[0]], sem_s).wait()
        pltpu.make_async_copy(ones_v, din_sh.at[idxd2.at[0]], sem_s).wait()

    def body(j, carry):
        _guarded(j >= _DEG_CAP, wait_two)
        pltpu.async_copy(ones_v, dout_sh.at[idxs2.at[j]], sem_s, add=True)
        pltpu.async_copy(ones_v, din_sh.at[idxd2.at[j]], sem_s, add=True)
        return carry

    lax.fori_loop(0, n_w, body, 0)
    for _ in range(_DEG_CAP):
        wait_two()

    plsc.subcore_barrier()

    @pl.when(s == 0)
    def _():
        pltpu.sync_copy(dout_sh, degp_hbm.at[c, 0])
        pltpu.sync_copy(din_sh, degp_hbm.at[c, 1])


_deg_call = functools.partial(
    pl.kernel,
    mesh=_MESH,
    out_type=jax.ShapeDtypeStruct((NC, 2, N_NODES), jnp.float32),
    scratch_types=[
        pltpu.VMEM((IDX_ROWS * CH,), jnp.int32),
        pltpu.VMEM((IDX_ROWS, CH), jnp.int32),
        pltpu.VMEM((IDX_ROWS, CH), jnp.int32),
        pltpu.VMEM((CH,), jnp.float32),
        pltpu.SemaphoreType.DMA,
        pltpu.SemaphoreType.DMA,
        pltpu.VMEM_SHARED((N_NODES,), jnp.float32),
        pltpu.VMEM_SHARED((N_NODES,), jnp.float32),
    ],
)(_deg_body)


def _scale_phase(d00, d01, d10, d11, feat_hbm, h1x_hbm, sout_hbm, sin_hbm,
                 rows_v, sv, dv, tv, c, s):
    lo = s * ROW_SLICE

    def load_deg(src, dst):
        pltpu.sync_copy(src.at[pl.ds(lo, ROW_SLICE)],
                        dst.at[pl.ds(0, ROW_SLICE)])

        @pl.when(s == NS - 1)
        def _():
            pltpu.sync_copy(src.at[pl.ds(ROW_SLICE * NS, ROW_REM)],
                            dst.at[pl.ds(ROW_SLICE, ROW_REM)])

    def store_scale(src, dst):
        pltpu.sync_copy(src.at[pl.ds(0, ROW_SLICE)],
                        dst.at[pl.ds(lo, ROW_SLICE)])

        @pl.when(s == NS - 1)
        def _():
            pltpu.sync_copy(src.at[pl.ds(ROW_SLICE, ROW_REM)],
                            dst.at[pl.ds(ROW_SLICE * NS, ROW_REM)])

    def rsqrt_buf(buf):
        def body(k, carry):
            sl = pl.ds(k * 16, 16)
            buf[sl] = _rsqrt16(buf[sl] + tv[sl])
            return carry
        lax.fori_loop(0, (ROW_SLICE + ROW_REM) // 16, body, 0)

    load_deg(d00, sv)
    load_deg(d10, tv)
    rsqrt_buf(sv)

    @pl.when(c == 0)
    def _():
        load_deg(d01, dv)
        load_deg(d11, tv)
        rsqrt_buf(dv)
        store_scale(sv, sout_hbm)
        store_scale(dv, sin_hbm)

    def scale_rows(nb, off):
        pltpu.sync_copy(feat_hbm.at[pl.ds(lo + off, nb)],
                        rows_v.at[0, pl.ds(0, nb)])

        def blk(k2, carry):
            s16 = sv[pl.ds(off + k2 * 16, 16)]
            for rr in range(16):
                bc = s16.at[jnp.full((16,), rr, jnp.int32)].get(
                    mode="promise_in_bounds")
                r = k2 * 16 + rr
                for cc in range(8):
                    sl = pl.ds(cc * 16, 16)
                    rows_v[0, r, sl] = rows_v[0, r, sl] * bc
            return carry

        lax.fori_loop(0, nb // 16, blk, 0)
        pltpu.sync_copy(rows_v.at[0, pl.ds(0, nb)],
                        h1x_hbm.at[c, pl.ds(lo + off, nb)])

    def batch(b, carry):
        scale_rows(CH, b * CH)
        return carry

    lax.fori_loop(0, ROW_SLICE // CH, batch, 0)
    scale_rows(ROW_SLICE - (ROW_SLICE // CH) * CH,
               (ROW_SLICE // CH) * CH)

    @pl.when(s == NS - 1)
    def _():
        scale_rows(ROW_REM, ROW_SLICE)


def _agg_main(h_src, edges_hbm, zeros_hbm, aggp_hbm,
              idxs_b, idxd_b, rows_v, sem_i, sem_g, sem_s, agg_sh,
              c, s, skip_zero_copy=False):
    w = s * NC + c
    start_w, n_w = _worker_range(w)

    def issue_idx(j, q):
        base = (start_w + j) * CH
        pltpu.async_copy(edges_hbm.at[0, pl.ds(base, CH)], idxs_b.at[q],
                         sem_i.at[q])
        pltpu.async_copy(edges_hbm.at[1, pl.ds(base, CH)], idxd_b.at[q],
                         sem_i.at[q])

    def wait_idx(q):
        pltpu.make_async_copy(edges_hbm.at[0, pl.ds(0, CH)], idxs_b.at[q],
                              sem_i.at[q]).wait()
        pltpu.make_async_copy(edges_hbm.at[0, pl.ds(0, CH)], idxd_b.at[q],
                              sem_i.at[q]).wait()

    def wait_scatter(r):
        pltpu.make_async_copy(rows_v.at[r], agg_sh.at[idxd_b.at[0]],
                              sem_s.at[r]).wait()

    def issue_gather(q, r):
        pltpu.async_copy(h_src.at[idxs_b.at[q, pl.ds(0, HCH)]],
                         rows_v.at[r, pl.ds(0, HCH)], sem_g.at[r])
        pltpu.async_copy(h_src.at[idxs_b.at[q, pl.ds(HCH, HCH)]],
                         rows_v.at[r, pl.ds(HCH, HCH)], sem_g.at[r])

    def wait_gather(r):
        pltpu.make_async_copy(h_src.at[idxs_b.at[0, pl.ds(0, HCH)]],
                              rows_v.at[r, pl.ds(0, HCH)], sem_g.at[r]).wait()
        pltpu.make_async_copy(h_src.at[idxs_b.at[0, pl.ds(0, HCH)]],
                              rows_v.at[r, pl.ds(0, HCH)], sem_g.at[r]).wait()

    def next_gather(j, u):
        r1 = (u + 1) % RB
        wait_idx((u + 1) % IB)
        issue_gather((u + 1) % IB, r1)

    def chunk_step(j, u):
        r = u % RB
        _guarded(j >= 1, wait_scatter, (u + 1) % RB)
        _guarded(j + RB < n_w, issue_idx, j + RB, (u + RB) % IB)
        _guarded(j + 1 < n_w, next_gather, j, u)
        wait_gather(r)
        pltpu.async_copy(rows_v.at[r], agg_sh.at[idxd_b.at[u]],
                         sem_s.at[r], add=True)

    for j0 in range(RB):
        issue_idx(j0, j0)
    _copy_row_slices(zeros_hbm, agg_sh, s)
    plsc.subcore_barrier()
    wait_idx(0)
    issue_gather(0, 0)

    def outer(t, carry):
        for u in range(IB):
            j = t * IB + u
            _guarded(j < n_w, chunk_step, j, u)
        return carry

    lax.fori_loop(0, N_OUTER, outer, 0)
    _guarded(lax.rem(n_w, 2) == 1, wait_scatter, 0)
    _guarded(lax.rem(n_w, 2) == 0, wait_scatter, 1)

    plsc.subcore_barrier()

    _copy_row_slices(agg_sh, aggp_hbm.at[c], s)


def _agg_body(h_hbm, edges_hbm, zeros_hbm, aggp_hbm,
              idxs_b, idxd_b, rows_v, sem_i, sem_g, sem_s, agg_sh):
    c = lax.axis_index("c")
    s = lax.axis_index("s")
    _agg_main(h_hbm, edges_hbm, zeros_hbm, aggp_hbm,
              idxs_b, idxd_b, rows_v, sem_i, sem_g, sem_s, agg_sh, c, s)


_agg_call = functools.partial(
    pl.kernel,
    mesh=_MESH,
    out_type=jax.ShapeDtypeStruct((NC, N_NODES, D_FEAT), jnp.float32),
    scratch_types=[
        pltpu.VMEM((IB, CH), jnp.int32),
        pltpu.VMEM((IB, CH), jnp.int32),
        pltpu.VMEM((RB, CH, D_FEAT), jnp.float32),
        pltpu.SemaphoreType.DMA((IB,)),
        pltpu.SemaphoreType.DMA((RB,)),
        pltpu.SemaphoreType.DMA((RB,)),
        pltpu.VMEM_SHARED((N_NODES, D_FEAT), jnp.float32),
    ],
)(_agg_body)


def _agg_pre_body(d00, d01, d10, d11, feat_hbm, edges_hbm, zeros_hbm,
                  aggp_hbm, h1x_hbm, sout_hbm, sin_hbm,
                  idxs_b, idxd_b, rows_v, sv, dv, tv,
                  sem_i, sem_g, sem_s, agg_sh):
    c = lax.axis_index("c")
    s = lax.axis_index("s")
    _scale_phase(d00, d01, d10, d11, feat_hbm, h1x_hbm, sout_hbm, sin_hbm,
                 rows_v, sv, dv, tv, c, s)
    _agg_main(h1x_hbm.at[c], edges_hbm, zeros_hbm, aggp_hbm,
              idxs_b, idxd_b, rows_v, sem_i, sem_g, sem_s, agg_sh, c, s)


_agg_pre_call = functools.partial(
    pl.kernel,
    mesh=_MESH,
    out_type=(
        jax.ShapeDtypeStruct((NC, N_NODES, D_FEAT), jnp.float32),
        jax.ShapeDtypeStruct((NC, N_NODES, D_FEAT), jnp.float32),
        jax.ShapeDtypeStruct((N_NODES,), jnp.float32),
        jax.ShapeDtypeStruct((N_NODES,), jnp.float32),
    ),
    scratch_types=[
        pltpu.VMEM((IB, CH), jnp.int32),
        pltpu.VMEM((IB, CH), jnp.int32),
        pltpu.VMEM((RB, CH, D_FEAT), jnp.float32),
        pltpu.VMEM((ROW_SLICE + ROW_REM,), jnp.float32),
        pltpu.VMEM((ROW_SLICE + ROW_REM,), jnp.float32),
        pltpu.VMEM((ROW_SLICE + ROW_REM,), jnp.float32),
        pltpu.SemaphoreType.DMA((IB,)),
        pltpu.SemaphoreType.DMA((RB,)),
        pltpu.SemaphoreType.DMA((RB,)),
        pltpu.VMEM_SHARED((N_NODES, D_FEAT), jnp.float32),
    ],
)(_agg_pre_body)


_BR = 1000


def _layer_body(scale_out, aggp_ref, scales_ref, w_ref, b_ref, out_ref):
    a = aggp_ref[0] + aggp_ref[1]
    x = a * scales_ref[:, 1:2]
    y = jnp.dot(x, w_ref[...], preferred_element_type=jnp.float32) + b_ref[...]
    z = jnp.maximum(y, 0.01 * y)
    if scale_out:
        z = z * scales_ref[:, 0:1]
    out_ref[...] = z


def _layer_call(aggp, scales, W, b2d, scale_out):
    return pl.pallas_call(
        functools.partial(_layer_body, scale_out),
        grid=(N_NODES // _BR,),
        in_specs=[
            pl.BlockSpec((NC, _BR, D_FEAT), lambda i: (0, i, 0)),
            pl.BlockSpec((_BR, 2), lambda i: (i, 0)),
            pl.BlockSpec((D_FEAT, D_FEAT), lambda i: (0, 0)),
            pl.BlockSpec((1, D_FEAT), lambda i: (0, 0)),
        ],
        out_specs=pl.BlockSpec((_BR, D_FEAT), lambda i: (i, 0)),
        out_shape=jax.ShapeDtypeStruct((N_NODES, D_FEAT), jnp.float32),
    )(aggp, scales, W, b2d)


def kernel(features, edge_index, W1, b1, W2, b2):
    edges = edge_index.astype(jnp.int32)
    zeros1 = jnp.zeros((N_NODES,), jnp.float32)
    zeros2 = jnp.zeros((N_NODES, D_FEAT), jnp.float32)

    degp = _deg_call(edges, zeros1)
    aggp1, _h1x, sout, sin = _agg_pre_call(
        degp[0, 0], degp[0, 1], degp[1, 0], degp[1, 1],
        features, edges, zeros2)
    s_col = jnp.stack([sout, sin], axis=1)
    h2 = _layer_call(aggp1, s_col, W1, b1.reshape(1, D_FEAT), True)

    aggp2 = _agg_call(h2, edges, zeros2)
    out = _layer_call(aggp2, s_col, W2, b2.reshape(1, D_FEAT), False)
    return out
